# baseline (speedup 1.0000x reference)
import functools

import jax
import jax.numpy as jnp
from jax import lax
from jax.experimental import pallas as pl
from jax.experimental.pallas import tpu as pltpu
from jax.experimental.pallas import tpu_sc as plsc

N = 4096
E = 131072
IN_C = 128
OUT_C = 32
PAD = 48
NC = 2
NS = 16
NW = NC * NS
EPT = E // NW
CH = 128
NCHUNK = EPT // CH
NACC = 8


def _tc_prep(x_ref, w_ref, a_ref, sup_ref, s2_ref):
    sup = jnp.dot(x_ref[...], w_ref[...], preferred_element_type=jnp.float32)
    sup_ref[...] = jnp.concatenate(
        [sup, jnp.zeros((N, PAD - OUT_C), jnp.float32)], axis=1)
    s2_ref[...] = lax.dot_general(
        a_ref[...], sup, (((1,), (1,)), ((), ())),
        preferred_element_type=jnp.float32,
        precision=lax.Precision.HIGHEST)


def _sc_edges(sup_hbm, src_hbm, dst_hbm, adj_hbm, s2_hbm, out_hbm,
              acc_sh, ssrc_v, sdst_v, zbuf_v, idxs_v, idxd_v, adj_v, rows_v, sem):
    cid = lax.axis_index("c")
    sid = lax.axis_index("s")
    wid = sid * NC + cid
    phase = sid // NACC
    accbase = (sid % NACC) * N

    zero16 = jnp.zeros((16,), jnp.float32)
    for r in range(CH):
        for k in range(PAD // 16):
            zbuf_v[r, 16 * k:16 * (k + 1)] = zero16
    zrows = NACC * N // NS

    def zero_it(i, carry):
        pltpu.sync_copy(zbuf_v, acc_sh.at[pl.ds(sid * zrows + i * CH, CH)])
        return carry

    lax.fori_loop(0, zrows // CH, zero_it, 0)
    pltpu.sync_copy(s2_hbm.at[0], ssrc_v)
    pltpu.sync_copy(s2_hbm.at[1], sdst_v)
    plsc.subcore_barrier()

    iota16 = lax.iota(jnp.int32, 16)
    col32 = jnp.full((16,), OUT_C, jnp.int32)

    def chunk(c, carry):
        base = wid * EPT + c * CH
        pltpu.sync_copy(src_hbm.at[pl.ds(base, CH)], idxs_v)
        pltpu.sync_copy(dst_hbm.at[pl.ds(base, CH)], idxd_v)
        pltpu.sync_copy(adj_hbm.at[pl.ds(base, CH)], adj_v)
        pltpu.async_copy(sup_hbm.at[idxd_v], rows_v, sem).wait()
        for g in range(CH // 16):
            sl = pl.ds(16 * g, 16)
            sidx = idxs_v[sl]
            ss = plsc.load_gather(ssrc_v, [sidx])
            sd = plsc.load_gather(sdst_v, [idxd_v[sl]])
            s = ss + sd
            w = jnp.exp(jnp.maximum(s, 0.2 * s))
            plsc.store_scatter(rows_v, [iota16 + 16 * g, col32], adj_v[sl])
            idxs_v[sl] = sidx + accbase
            for j in range(16):
                e = 16 * g + j
                we = w[j]
                rows_v[e, 0:16] = rows_v[e, 0:16] * we
                rows_v[e, 16:32] = rows_v[e, 16:32] * we
        @pl.when(phase == 0)
        def _():
            pltpu.sync_copy(rows_v, acc_sh.at[idxs_v], add=True)

        plsc.subcore_barrier()

        @pl.when(phase == 1)
        def _():
            pltpu.sync_copy(rows_v, acc_sh.at[idxs_v], add=True)

        plsc.subcore_barrier()
        return carry

    lax.fori_loop(0, NCHUNK, chunk, 0)
    plsc.subcore_barrier()

    rows_per_tile = N // NS
    for h in range(rows_per_tile // CH):
        off = sid * rows_per_tile + h * CH
        for g in range(CH // 16):
            idxs_v[pl.ds(16 * g, 16)] = iota16 + (16 * g + off)
        for j in range(1, NACC):
            pltpu.sync_copy(acc_sh.at[pl.ds(j * N + off, CH)], rows_v)
            pltpu.sync_copy(rows_v, acc_sh.at[idxs_v], add=True)
        pltpu.sync_copy(acc_sh.at[pl.ds(off, CH)],
                        out_hbm.at[cid, pl.ds(off, CH)])


def _tc_finish(parts_ref, out_ref):
    p = parts_ref[0] + parts_ref[1]
    r = p[:, OUT_C:OUT_C + 1]
    out_ref[...] = p[:, :OUT_C] / jnp.where(r == 0.0, 1.0, r)


@jax.jit
def kernel(x, edge_index, adj_values, weight, attention):
    attn2 = attention.reshape(2, OUT_C)
    sup_pad, s2 = pl.pallas_call(
        _tc_prep,
        out_shape=[
            jax.ShapeDtypeStruct((N, PAD), jnp.float32),
            jax.ShapeDtypeStruct((2, N), jnp.float32),
        ],
    )(x, weight, attn2)

    src = edge_index[0]
    dst = edge_index[1]

    sc_call = functools.partial(
        pl.kernel,
        mesh=plsc.VectorSubcoreMesh(core_axis_name="c", subcore_axis_name="s"),
        out_type=jax.ShapeDtypeStruct((NC, N, PAD), jnp.float32),
        scratch_types=[
            pltpu.VMEM_SHARED((NACC * N, PAD), jnp.float32),
            pltpu.VMEM((N,), jnp.float32),
            pltpu.VMEM((N,), jnp.float32),
            pltpu.VMEM((CH, PAD), jnp.float32),
            pltpu.VMEM((CH,), jnp.int32),
            pltpu.VMEM((CH,), jnp.int32),
            pltpu.VMEM((CH,), jnp.float32),
            pltpu.VMEM((CH, PAD), jnp.float32),
            pltpu.SemaphoreType.DMA,
        ],
        compiler_params=pltpu.CompilerParams(
            needs_layout_passes=False, use_tc_tiling_on_sc=False),
    )(_sc_edges)
    parts = sc_call(sup_pad, src, dst, adj_values, s2)

    out = pl.pallas_call(
        _tc_finish,
        out_shape=jax.ShapeDtypeStruct((N, OUT_C), jnp.float32),
    )(parts)
    return out.reshape(N, 1, OUT_C)


# speedup vs baseline: 1.3543x; 1.3543x over previous
import functools

import jax
import jax.numpy as jnp
from jax import lax
from jax.experimental import pallas as pl
from jax.experimental.pallas import tpu as pltpu
from jax.experimental.pallas import tpu_sc as plsc

N = 4096
E = 131072
IN_C = 128
OUT_C = 32
NC = 2
NS = 16
NW = NC * NS
EPT = E // NW
CH = 128
NCHUNK = EPT // CH
NACC = 8


def _tc_prep(x_ref, w_ref, a_ref, sup_ref, s2_ref):
    sup = jnp.dot(x_ref[...], w_ref[...], preferred_element_type=jnp.float32)
    sup_ref[...] = sup
    s2_ref[...] = lax.dot_general(
        a_ref[...], sup, (((1,), (1,)), ((), ())),
        preferred_element_type=jnp.float32,
        precision=lax.Precision.HIGHEST)


def _sc_edges(sup_hbm, src_hbm, dst_hbm, adj_hbm, s2_hbm, out_hbm, rs_hbm,
              acc_sh, rs_sh, ssrc_v, sdst_v, zbuf_v, zrs_v, src_t, dst_t,
              adj_t, idxs_v, rows_v, rsbuf_v, sem):
    cid = lax.axis_index("c")
    sid = lax.axis_index("s")
    wid = sid * NC + cid
    phase = sid // NACC
    accbase = (sid % NACC) * N

    zero16 = jnp.zeros((16,), jnp.float32)
    for r in range(CH):
        for k in range(OUT_C // 16):
            zbuf_v[r, 16 * k:16 * (k + 1)] = zero16
    for k in range(CH // 16):
        zrs_v[16 * k:16 * (k + 1)] = zero16
    zrows = NACC * N // NS

    def zero_it(i, carry):
        pltpu.sync_copy(zbuf_v, acc_sh.at[pl.ds(sid * zrows + i * CH, CH)])
        pltpu.sync_copy(zrs_v, rs_sh.at[pl.ds(sid * zrows + i * CH, CH)])
        return carry

    lax.fori_loop(0, zrows // CH, zero_it, 0)
    pltpu.sync_copy(s2_hbm.at[0], ssrc_v)
    pltpu.sync_copy(s2_hbm.at[1], sdst_v)
    pltpu.sync_copy(src_hbm.at[pl.ds(wid * EPT, EPT)], src_t)
    pltpu.sync_copy(dst_hbm.at[pl.ds(wid * EPT, EPT)], dst_t)
    pltpu.sync_copy(adj_hbm.at[pl.ds(wid * EPT, EPT)], adj_t)
    plsc.subcore_barrier()

    def chunk(c, carry):
        base = c * CH
        pltpu.async_copy(sup_hbm.at[dst_t.at[pl.ds(base, CH)]], rows_v,
                         sem).wait()
        for g in range(CH // 16):
            sl = pl.ds(16 * g, 16)
            sidx = src_t[pl.ds(base + 16 * g, 16)]
            ss = plsc.load_gather(ssrc_v, [sidx])
            sd = plsc.load_gather(sdst_v, [dst_t[pl.ds(base + 16 * g, 16)]])
            s = ss + sd
            w = jnp.exp(jnp.maximum(s, 0.2 * s))
            idxs_v[sl] = sidx + accbase
            for j in range(16):
                e = 16 * g + j
                we = w[j]
                rows_v[e, 0:16] = rows_v[e, 0:16] * we
                rows_v[e, 16:32] = rows_v[e, 16:32] * we
        @pl.when(phase == 0)
        def _():
            pltpu.sync_copy(rows_v, acc_sh.at[idxs_v], add=True)
            pltpu.sync_copy(adj_t.at[pl.ds(base, CH)], rs_sh.at[idxs_v],
                            add=True)

        plsc.subcore_barrier()

        @pl.when(phase == 1)
        def _():
            pltpu.sync_copy(rows_v, acc_sh.at[idxs_v], add=True)
            pltpu.sync_copy(adj_t.at[pl.ds(base, CH)], rs_sh.at[idxs_v],
                            add=True)

        plsc.subcore_barrier()
        return carry

    lax.fori_loop(0, NCHUNK, chunk, 0)
    plsc.subcore_barrier()

    iota16 = lax.iota(jnp.int32, 16)
    rows_per_tile = N // NS
    for h in range(rows_per_tile // CH):
        off = sid * rows_per_tile + h * CH
        for g in range(CH // 16):
            idxs_v[pl.ds(16 * g, 16)] = iota16 + (16 * g + off)
        for j in range(1, NACC):
            pltpu.sync_copy(acc_sh.at[pl.ds(j * N + off, CH)], rows_v)
            pltpu.sync_copy(rows_v, acc_sh.at[idxs_v], add=True)
            pltpu.sync_copy(rs_sh.at[pl.ds(j * N + off, CH)], rsbuf_v)
            pltpu.sync_copy(rsbuf_v, rs_sh.at[idxs_v], add=True)
        pltpu.sync_copy(acc_sh.at[pl.ds(off, CH)],
                        out_hbm.at[cid, pl.ds(off, CH)])
        pltpu.sync_copy(rs_sh.at[pl.ds(off, CH)],
                        rs_hbm.at[cid, pl.ds(off, CH)])


def _tc_finish(parts_ref, rs_ref, out_ref):
    p = parts_ref[0] + parts_ref[1]
    r = (rs_ref[0] + rs_ref[1])[:, None]
    out_ref[...] = p / jnp.where(r == 0.0, 1.0, r)


@jax.jit
def kernel(x, edge_index, adj_values, weight, attention):
    attn2 = attention.reshape(2, OUT_C)
    sup, s2 = pl.pallas_call(
        _tc_prep,
        out_shape=[
            jax.ShapeDtypeStruct((N, OUT_C), jnp.float32),
            jax.ShapeDtypeStruct((2, N), jnp.float32),
        ],
    )(x, weight, attn2)

    src = edge_index[0]
    dst = edge_index[1]

    sc_call = functools.partial(
        pl.kernel,
        mesh=plsc.VectorSubcoreMesh(core_axis_name="c", subcore_axis_name="s"),
        out_type=[
            jax.ShapeDtypeStruct((NC, N, OUT_C), jnp.float32),
            jax.ShapeDtypeStruct((NC, N), jnp.float32),
        ],
        scratch_types=[
            pltpu.VMEM_SHARED((NACC * N, OUT_C), jnp.float32),
            pltpu.VMEM_SHARED((NACC * N,), jnp.float32),
            pltpu.VMEM((N,), jnp.float32),
            pltpu.VMEM((N,), jnp.float32),
            pltpu.VMEM((CH, OUT_C), jnp.float32),
            pltpu.VMEM((CH,), jnp.float32),
            pltpu.VMEM((EPT,), jnp.int32),
            pltpu.VMEM((EPT,), jnp.int32),
            pltpu.VMEM((EPT,), jnp.float32),
            pltpu.VMEM((CH,), jnp.int32),
            pltpu.VMEM((CH, OUT_C), jnp.float32),
            pltpu.VMEM((CH,), jnp.float32),
            pltpu.SemaphoreType.DMA,
        ],
        compiler_params=pltpu.CompilerParams(
            needs_layout_passes=False, use_tc_tiling_on_sc=False),
    )(_sc_edges)
    parts, rs = sc_call(sup, src, dst, adj_values, s2)

    out = pl.pallas_call(
        _tc_finish,
        out_shape=jax.ShapeDtypeStruct((N, OUT_C), jnp.float32),
    )(parts, rs)
    return out.reshape(N, 1, OUT_C)


# speedup vs baseline: 1.4957x; 1.1044x over previous
import functools

import jax
import jax.numpy as jnp
from jax import lax
from jax.experimental import pallas as pl
from jax.experimental.pallas import tpu as pltpu
from jax.experimental.pallas import tpu_sc as plsc

N = 4096
E = 131072
IN_C = 128
OUT_C = 32
NC = 2
NS = 16
NW = NC * NS
EPT = E // NW
CH = 128
NCHUNK = EPT // CH
NACC = 8


def _tc_prep(x_ref, w_ref, a_ref, sup_ref, s2_ref):
    sup = jnp.dot(x_ref[...], w_ref[...], preferred_element_type=jnp.float32)
    sup_ref[...] = sup
    s2_ref[...] = lax.dot_general(
        a_ref[...], sup, (((1,), (1,)), ((), ())),
        preferred_element_type=jnp.float32,
        precision=lax.Precision.HIGHEST)


def _sc_edges(sup_hbm, src_hbm, dst_hbm, adj_hbm, s2_hbm, out_hbm, rs_hbm,
              acc_sh, rs_sh, ssrc_v, sdst_v, zbuf_v, zrs_v, src_t, dst_t,
              adj_t, idxs_v, idxs2_v, rows_v, rows2_v, rsbuf_v, sem, sem2):
    cid = lax.axis_index("c")
    sid = lax.axis_index("s")
    wid = sid * NC + cid
    phase = sid // NACC
    accbase = (sid % NACC) * N

    zero16 = jnp.zeros((16,), jnp.float32)
    for r in range(CH):
        for k in range(OUT_C // 16):
            zbuf_v[r, 16 * k:16 * (k + 1)] = zero16
    for k in range(CH // 16):
        zrs_v[16 * k:16 * (k + 1)] = zero16
    zrows = NACC * N // NS

    def zero_it(i, carry):
        pltpu.sync_copy(zbuf_v, acc_sh.at[pl.ds(sid * zrows + i * CH, CH)])
        pltpu.sync_copy(zrs_v, rs_sh.at[pl.ds(sid * zrows + i * CH, CH)])
        return carry

    lax.fori_loop(0, zrows // CH, zero_it, 0)
    pltpu.sync_copy(s2_hbm.at[0], ssrc_v)
    pltpu.sync_copy(s2_hbm.at[1], sdst_v)
    pltpu.sync_copy(src_hbm.at[pl.ds(wid * EPT, EPT)], src_t)
    pltpu.sync_copy(dst_hbm.at[pl.ds(wid * EPT, EPT)], dst_t)
    pltpu.sync_copy(adj_hbm.at[pl.ds(wid * EPT, EPT)], adj_t)
    plsc.subcore_barrier()

    def compute(base, rows, idxs):
        for g in range(CH // 16):
            sl = pl.ds(16 * g, 16)
            sidx = src_t[pl.ds(base + 16 * g, 16)]
            ss = plsc.load_gather(ssrc_v, [sidx])
            sd = plsc.load_gather(sdst_v, [dst_t[pl.ds(base + 16 * g, 16)]])
            s = ss + sd
            w = jnp.exp(jnp.maximum(s, 0.2 * s))
            idxs[sl] = sidx + accbase
            for j in range(16):
                e = 16 * g + j
                we = w[j]
                rows[e, 0:16] = rows[e, 0:16] * we
                rows[e, 16:32] = rows[e, 16:32] * we

    def scatter(base_a, base_b):
        pltpu.sync_copy(rows_v, acc_sh.at[idxs_v], add=True)
        pltpu.sync_copy(rows2_v, acc_sh.at[idxs2_v], add=True)
        pltpu.sync_copy(adj_t.at[pl.ds(base_a, CH)], rs_sh.at[idxs_v],
                        add=True)
        pltpu.sync_copy(adj_t.at[pl.ds(base_b, CH)], rs_sh.at[idxs2_v],
                        add=True)

    def chunk(c, carry):
        base_a = 2 * c * CH
        base_b = base_a + CH
        cp_a = pltpu.async_copy(sup_hbm.at[dst_t.at[pl.ds(base_a, CH)]],
                                rows_v, sem)
        cp_b = pltpu.async_copy(sup_hbm.at[dst_t.at[pl.ds(base_b, CH)]],
                                rows2_v, sem2)
        cp_a.wait()
        compute(base_a, rows_v, idxs_v)
        cp_b.wait()
        compute(base_b, rows2_v, idxs2_v)
        @pl.when(phase == 0)
        def _():
            scatter(base_a, base_b)

        plsc.subcore_barrier()

        @pl.when(phase == 1)
        def _():
            scatter(base_a, base_b)

        plsc.subcore_barrier()
        return carry

    lax.fori_loop(0, NCHUNK // 2, chunk, 0)
    plsc.subcore_barrier()

    iota16 = lax.iota(jnp.int32, 16)
    rows_per_tile = N // NS
    for h in range(rows_per_tile // CH):
        off = sid * rows_per_tile + h * CH
        for g in range(CH // 16):
            idxs_v[pl.ds(16 * g, 16)] = iota16 + (16 * g + off)
        for j in range(1, NACC):
            pltpu.sync_copy(acc_sh.at[pl.ds(j * N + off, CH)], rows_v)
            pltpu.sync_copy(rows_v, acc_sh.at[idxs_v], add=True)
            pltpu.sync_copy(rs_sh.at[pl.ds(j * N + off, CH)], rsbuf_v)
            pltpu.sync_copy(rsbuf_v, rs_sh.at[idxs_v], add=True)
        pltpu.sync_copy(acc_sh.at[pl.ds(off, CH)],
                        out_hbm.at[cid, pl.ds(off, CH)])
        pltpu.sync_copy(rs_sh.at[pl.ds(off, CH)],
                        rs_hbm.at[cid, pl.ds(off, CH)])


def _tc_finish(parts_ref, rs_ref, out_ref):
    p = parts_ref[0] + parts_ref[1]
    r = (rs_ref[0] + rs_ref[1])[:, None]
    out_ref[...] = p / jnp.where(r == 0.0, 1.0, r)


@jax.jit
def kernel(x, edge_index, adj_values, weight, attention):
    attn2 = attention.reshape(2, OUT_C)
    sup, s2 = pl.pallas_call(
        _tc_prep,
        out_shape=[
            jax.ShapeDtypeStruct((N, OUT_C), jnp.float32),
            jax.ShapeDtypeStruct((2, N), jnp.float32),
        ],
    )(x, weight, attn2)

    src = edge_index[0]
    dst = edge_index[1]

    sc_call = functools.partial(
        pl.kernel,
        mesh=plsc.VectorSubcoreMesh(core_axis_name="c", subcore_axis_name="s"),
        out_type=[
            jax.ShapeDtypeStruct((NC, N, OUT_C), jnp.float32),
            jax.ShapeDtypeStruct((NC, N), jnp.float32),
        ],
        scratch_types=[
            pltpu.VMEM_SHARED((NACC * N, OUT_C), jnp.float32),
            pltpu.VMEM_SHARED((NACC * N,), jnp.float32),
            pltpu.VMEM((N,), jnp.float32),
            pltpu.VMEM((N,), jnp.float32),
            pltpu.VMEM((CH, OUT_C), jnp.float32),
            pltpu.VMEM((CH,), jnp.float32),
            pltpu.VMEM((EPT,), jnp.int32),
            pltpu.VMEM((EPT,), jnp.int32),
            pltpu.VMEM((EPT,), jnp.float32),
            pltpu.VMEM((CH,), jnp.int32),
            pltpu.VMEM((CH,), jnp.int32),
            pltpu.VMEM((CH, OUT_C), jnp.float32),
            pltpu.VMEM((CH, OUT_C), jnp.float32),
            pltpu.VMEM((CH,), jnp.float32),
            pltpu.SemaphoreType.DMA,
            pltpu.SemaphoreType.DMA,
        ],
        compiler_params=pltpu.CompilerParams(
            needs_layout_passes=False, use_tc_tiling_on_sc=False),
    )(_sc_edges)
    parts, rs = sc_call(sup, src, dst, adj_values, s2)

    out = pl.pallas_call(
        _tc_finish,
        out_shape=jax.ShapeDtypeStruct((N, OUT_C), jnp.float32),
    )(parts, rs)
    return out.reshape(N, 1, OUT_C)
